# baseline (speedup 1.0000x reference)
import jax
import jax.numpy as jnp
from jax import lax
from jax.experimental import pallas as pl
from jax.experimental.pallas import tpu as pltpu

N = 1024
HID = 128
DE = 8
BLK = 128
NB = N // BLK
CH = 16
NCH = BLK // CH
JS = 128
N_REL = 12
N_ROLE = 16
N_CNT = 9


def _fused_kernel(h_ref, eft_ref, adj_ref,
                  wcs_ref, wct_ref, wrs_ref, wrt_ref,
                  w1bd_ref, b1big_ref, w2big_ref, b2pair_ref,
                  w1stack_ref, b1col_ref, w2col_ref,
                  relw1_ref, relb1_ref, relw2_ref, relb2_ref,
                  attnk_ref, attnq_ref,
                  gpw_ref, gpb_ref, rolew_ref, roleb_ref,
                  cntw_ref, cntb_ref, noopw_ref, noopb_ref,
                  create_ref, remove_ref, rel_ref, role_ref,
                  cnt_ref, noop_ref, g_ref,
                  smax_ref, sidx_ref, fec_ref):
    b = pl.program_id(0)
    scale = 1.0 / jnp.sqrt(jnp.float32(HID))
    h = h_ref[:, :]
    hb = h_ref[pl.ds(b * BLK, BLK), :]

    ac = jnp.dot(hb, wcs_ref[:, :], preferred_element_type=jnp.float32)
    bc = jnp.dot(h, wct_ref[:, :], preferred_element_type=jnp.float32)
    sc = lax.dot_general(ac, bc, (((1,), (1,)), ((), ())),
                         preferred_element_type=jnp.float32) * scale
    ar = jnp.dot(hb, wrs_ref[:, :], preferred_element_type=jnp.float32)
    br = jnp.dot(h, wrt_ref[:, :], preferred_element_type=jnp.float32)
    sr = lax.dot_general(ar, br, (((1,), (1,)), ((), ())),
                         preferred_element_type=jnp.float32) * scale
    create_ref[:, :] = sc
    remove_ref[:, :] = sr

    b2c = b2pair_ref[0, 0]
    b2r = b2pair_ref[0, 1]

    def chunk_body(c, carry):
        base = c * CH * DE
        xm = eft_ref[pl.ds(base, CH * DE), JS:N]
        hid = jnp.dot(w1bd_ref[:, :], xm,
                      preferred_element_type=jnp.float32) + b1big_ref[:, :]
        hid = jnp.maximum(hid, 0.0) * w2big_ref[:, :]
        srows = hid.reshape(2 * CH, HID, N - JS).sum(axis=1)
        create_ref[pl.ds(c * CH, CH), JS:N] += srows[0:CH] + b2c
        remove_ref[pl.ds(c * CH, CH), JS:N] += srows[CH:2 * CH] + b2r

        ceb_rows = []
        reb_rows = []
        for r in range(CH):
            g = eft_ref[pl.ds(base + r * DE, DE), 0:JS]
            hv = w1stack_ref[0:2 * HID, :] * g[0:1, :]
            for d in range(1, DE):
                hv = hv + w1stack_ref[pl.ds(d * 2 * HID, 2 * HID), :] * g[d:d + 1, :]
            hv = jnp.maximum(hv + b1col_ref[:, :], 0.0) * w2col_ref[:, :]
            ceb_rows.append(jnp.sum(hv[0:HID], axis=0, keepdims=True))
            reb_rows.append(jnp.sum(hv[HID:2 * HID], axis=0, keepdims=True))
        create_ref[pl.ds(c * CH, CH), 0:JS] += (
            jnp.concatenate(ceb_rows, axis=0) + b2c)
        remove_ref[pl.ds(c * CH, CH), 0:JS] += (
            jnp.concatenate(reb_rows, axis=0) + b2r)
        return carry

    lax.fori_loop(0, NCH, chunk_body, 0)

    rows_l = lax.broadcasted_iota(jnp.int32, (BLK, N), 0)
    cols = lax.broadcasted_iota(jnp.int32, (BLK, N), 1)
    rows_g = rows_l + b * BLK
    valid = (adj_ref[:, :] <= 0.5) & (rows_g != cols)
    mc = jnp.where(valid, create_ref[:, :], -jnp.inf)
    vm = jnp.max(mc)
    flat = rows_g * N + cols
    cand = jnp.where(mc == vm, flat, jnp.int32(2**31 - 1))
    li = jnp.min(cand)

    @pl.when((vm > smax_ref[0]) | (b == 0))
    def _update():
        smax_ref[0] = vm
        ii = li // N
        jj = li - ii * N
        sidx_ref[0] = ii
        sidx_ref[1] = jj
        i_loc = ii - b * BLK
        slab = eft_ref[pl.ds(i_loc * DE, DE), :]
        colmask = lax.broadcasted_iota(jnp.int32, (DE, N), 1) == jj
        fec_ref[:, :] = jnp.sum(jnp.where(colmask, slab, 0.0), axis=1,
                                keepdims=True)

    @pl.when(b == NB - 1)
    def _finish():
        ii = sidx_ref[0]
        jj = sidx_ref[1]
        hi = h_ref[pl.ds(ii, 1), :]
        hj = h_ref[pl.ds(jj, 1), :]
        w1a = relw1_ref[0:HID, :]
        w1b = relw1_ref[HID:2 * HID, :]
        w1c = relw1_ref[2 * HID:3 * HID, :]
        w1d = relw1_ref[3 * HID:4 * HID, :]
        w1e = relw1_ref[4 * HID:4 * HID + DE, :]
        ph = (jnp.dot(hi, w1a, preferred_element_type=jnp.float32)
              + jnp.dot(hj, w1b, preferred_element_type=jnp.float32)
              + jnp.dot(hi * hj, w1c, preferred_element_type=jnp.float32)
              + jnp.dot(jnp.abs(hi - hj), w1d, preferred_element_type=jnp.float32)
              + lax.dot_general(fec_ref[:, :], w1e, (((0,), (0,)), ((), ())),
                                preferred_element_type=jnp.float32)
              + relb1_ref[:, :])
        rh = jnp.maximum(ph, 0.0)
        rel_ref[:, :] = jnp.dot(rh, relw2_ref[:, :],
                                preferred_element_type=jnp.float32) + relb2_ref[:, :]

        keys = jnp.dot(h, attnk_ref[:, :], preferred_element_type=jnp.float32)
        al = jnp.dot(keys, attnq_ref[:, :],
                     preferred_element_type=jnp.float32) * scale
        m = jnp.max(al)
        e = jnp.exp(al - m)
        attn = e / jnp.sum(e)
        pooled = jnp.sum(attn * h, axis=0, keepdims=True)
        g = jnp.dot(pooled, gpw_ref[:, :],
                    preferred_element_type=jnp.float32) + gpb_ref[:, :]
        g_ref[:, :] = g
        role_ref[:, :] = jnp.dot(g, rolew_ref[:, :],
                                 preferred_element_type=jnp.float32) + roleb_ref[:, :]
        cnt_ref[:, :] = jnp.dot(g, cntw_ref[:, :],
                                preferred_element_type=jnp.float32) + cntb_ref[:, :]
        noop_ref[:, :] = jnp.dot(g, noopw_ref[:, :],
                                 preferred_element_type=jnp.float32) + noopb_ref[:, :]


def _full(shape):
    zeros = (0,) * len(shape)
    return pl.BlockSpec(shape, lambda b, z=zeros: z)


@jax.jit
def _run(node_embeddings, ef_t, adjacency, wcs, wct, wrs, wrt,
         w1bd, b1big, w2big, b2pair, w1stack, b1col, w2col,
         relw1, relb1, relw2, relb2,
         attnk, attnq, gpw, gpb, rolew, roleb, cntw, cntb, noopw, noopb):
    out_shapes = (
        jax.ShapeDtypeStruct((N, N), jnp.float32),
        jax.ShapeDtypeStruct((N, N), jnp.float32),
        jax.ShapeDtypeStruct((1, N_REL), jnp.float32),
        jax.ShapeDtypeStruct((1, N_ROLE), jnp.float32),
        jax.ShapeDtypeStruct((1, N_CNT), jnp.float32),
        jax.ShapeDtypeStruct((1, 1), jnp.float32),
        jax.ShapeDtypeStruct((1, HID), jnp.float32),
    )
    in_specs = [
        _full((N, HID)),
        pl.BlockSpec((BLK * DE, N), lambda b: (b, 0)),
        pl.BlockSpec((BLK, N), lambda b: (b, 0)),
        _full((HID, HID)), _full((HID, HID)), _full((HID, HID)), _full((HID, HID)),
        _full((2 * CH * HID, CH * DE)), _full((2 * CH * HID, 1)),
        _full((2 * CH * HID, 1)), _full((1, 2)),
        _full((DE * 2 * HID, 1)), _full((2 * HID, 1)), _full((2 * HID, 1)),
        _full((4 * HID + DE, HID)), _full((1, HID)),
        _full((HID, N_REL)), _full((1, N_REL)),
        _full((HID, HID)), _full((HID, 1)),
        _full((HID, HID)), _full((1, HID)),
        _full((HID, N_ROLE)), _full((1, N_ROLE)),
        _full((HID, N_CNT)), _full((1, N_CNT)),
        _full((HID, 1)), _full((1, 1)),
    ]
    out_specs = (
        pl.BlockSpec((BLK, N), lambda b: (b, 0)),
        pl.BlockSpec((BLK, N), lambda b: (b, 0)),
        _full((1, N_REL)), _full((1, N_ROLE)), _full((1, N_CNT)),
        _full((1, 1)), _full((1, HID)),
    )
    return pl.pallas_call(
        _fused_kernel,
        grid=(NB,),
        in_specs=in_specs,
        out_specs=out_specs,
        out_shape=out_shapes,
        scratch_shapes=[
            pltpu.SMEM((1,), jnp.float32),
            pltpu.SMEM((2,), jnp.int32),
            pltpu.VMEM((DE, 1), jnp.float32),
        ],
        compiler_params=pltpu.CompilerParams(
            dimension_semantics=("arbitrary",),
        ),
    )(node_embeddings, ef_t, adjacency, wcs, wct, wrs, wrt,
      w1bd, b1big, w2big, b2pair, w1stack, b1col, w2col,
      relw1, relb1, relw2, relb2,
      attnk, attnq, gpw, gpb, rolew, roleb, cntw, cntb, noopw, noopb)


def kernel(node_embeddings, edge_features, adjacency, params):
    p = params
    ef_t = jnp.transpose(edge_features,
                         (0, 2, 1)).reshape(N * DE, N)
    eye_ch = jnp.eye(CH, dtype=jnp.float32)
    w1bd = jnp.concatenate([
        jnp.kron(eye_ch, p["ceb_w1"].T),
        jnp.kron(eye_ch, p["reb_w1"].T),
    ], axis=0)
    b1big = jnp.concatenate([
        jnp.tile(p["ceb_b1"], CH), jnp.tile(p["reb_b1"], CH),
    ]).reshape(2 * CH * HID, 1)
    w2big = jnp.concatenate([
        jnp.tile(p["ceb_w2"][:, 0], CH), jnp.tile(p["reb_w2"][:, 0], CH),
    ]).reshape(2 * CH * HID, 1)
    b2pair = jnp.stack([p["ceb_b2"][0], p["reb_b2"][0]]).reshape(1, 2)
    w1cat2 = jnp.concatenate([p["ceb_w1"].T, p["reb_w1"].T], axis=0)
    w1stack = w1cat2.T.reshape(DE * 2 * HID, 1)
    b1col = jnp.concatenate([p["ceb_b1"], p["reb_b1"]]).reshape(2 * HID, 1)
    w2col = jnp.concatenate([p["ceb_w2"][:, 0], p["reb_w2"][:, 0]]).reshape(2 * HID, 1)

    outs = _run(node_embeddings, ef_t, adjacency,
                p["W_cs"], p["W_ct"], p["W_rs"], p["W_rt"],
                w1bd, b1big, w2big, b2pair, w1stack, b1col, w2col,
                p["rel_w1"], p["rel_b1"].reshape(1, HID),
                p["rel_w2"], p["rel_b2"].reshape(1, N_REL),
                p["attn_k"], p["attn_q"].reshape(HID, 1),
                p["gp_w"], p["gp_b"].reshape(1, HID),
                p["role_w"], p["role_b"].reshape(1, N_ROLE),
                p["cnt_w"], p["cnt_b"].reshape(1, N_CNT),
                p["noop_w"], p["noop_b"].reshape(1, 1))
    create, remove, rel, role, cnt, noop, g = outs
    return (create, remove, rel.reshape(N_REL), role.reshape(N_ROLE),
            cnt.reshape(N_CNT), noop.reshape(1), g.reshape(HID))


# speedup vs baseline: 1.4123x; 1.4123x over previous
import jax
import jax.numpy as jnp
from jax import lax
from jax.experimental import pallas as pl
from jax.experimental.pallas import tpu as pltpu

N = 1024
HID = 128
DE = 8
BLK = 128
NB = N // BLK
CH = 16
NCH = BLK // CH
N_REL = 12
N_ROLE = 16
N_CNT = 9


def _fused_kernel(h_ref, eft_ref, adj_ref,
                  wcs_ref, wct_ref, wrs_ref, wrt_ref,
                  w1bd_ref, b1big_ref, w2big_ref, b2pair_ref,
                  relw1_ref, relb1_ref, relw2_ref, relb2_ref,
                  attnk_ref, attnq_ref,
                  gpw_ref, gpb_ref, rolew_ref, roleb_ref,
                  cntw_ref, cntb_ref, noopw_ref, noopb_ref,
                  create_ref, remove_ref, rel_ref, role_ref,
                  cnt_ref, noop_ref, g_ref,
                  smax_ref, sidx_ref, fec_ref):
    b = pl.program_id(0)
    scale = 1.0 / jnp.sqrt(jnp.float32(HID))
    h = h_ref[:, :]
    hb = h_ref[pl.ds(b * BLK, BLK), :]

    ac = jnp.dot(hb, wcs_ref[:, :], preferred_element_type=jnp.float32)
    bc = jnp.dot(h, wct_ref[:, :], preferred_element_type=jnp.float32)
    sc = lax.dot_general(ac, bc, (((1,), (1,)), ((), ())),
                         preferred_element_type=jnp.float32) * scale
    ar = jnp.dot(hb, wrs_ref[:, :], preferred_element_type=jnp.float32)
    br = jnp.dot(h, wrt_ref[:, :], preferred_element_type=jnp.float32)
    sr = lax.dot_general(ar, br, (((1,), (1,)), ((), ())),
                         preferred_element_type=jnp.float32) * scale
    create_ref[:, :] = sc
    remove_ref[:, :] = sr

    b2c = b2pair_ref[0, 0]
    b2r = b2pair_ref[0, 1]

    def chunk_body(c, carry):
        x = eft_ref[pl.ds(c * CH * DE, CH * DE), :]
        hid = jnp.dot(w1bd_ref[:, :], x,
                      preferred_element_type=jnp.float32) + b1big_ref[:, :]
        hid = jnp.maximum(hid, 0.0) * w2big_ref[:, :]
        srows = hid.reshape(2 * CH, HID, N).sum(axis=1)
        create_ref[pl.ds(c * CH, CH), :] += srows[0:CH] + b2c
        remove_ref[pl.ds(c * CH, CH), :] += srows[CH:2 * CH] + b2r
        return carry

    lax.fori_loop(0, NCH, chunk_body, 0)

    rows_l = lax.broadcasted_iota(jnp.int32, (BLK, N), 0)
    cols = lax.broadcasted_iota(jnp.int32, (BLK, N), 1)
    rows_g = rows_l + b * BLK
    valid = (adj_ref[:, :] <= 0.5) & (rows_g != cols)
    mc = jnp.where(valid, create_ref[:, :], -jnp.inf)
    vm = jnp.max(mc)
    flat = rows_g * N + cols
    cand = jnp.where(mc == vm, flat, jnp.int32(2**31 - 1))
    li = jnp.min(cand)

    @pl.when((vm > smax_ref[0]) | (b == 0))
    def _update():
        smax_ref[0] = vm
        ii = li // N
        jj = li - ii * N
        sidx_ref[0] = ii
        sidx_ref[1] = jj
        i_loc = ii - b * BLK
        slab = eft_ref[pl.ds(i_loc * DE, DE), :]
        colmask = lax.broadcasted_iota(jnp.int32, (DE, N), 1) == jj
        fec_ref[:, :] = jnp.sum(jnp.where(colmask, slab, 0.0), axis=1,
                                keepdims=True)

    @pl.when(b == NB - 1)
    def _finish():
        ii = sidx_ref[0]
        jj = sidx_ref[1]
        hi = h_ref[pl.ds(ii, 1), :]
        hj = h_ref[pl.ds(jj, 1), :]
        w1a = relw1_ref[0:HID, :]
        w1b = relw1_ref[HID:2 * HID, :]
        w1c = relw1_ref[2 * HID:3 * HID, :]
        w1d = relw1_ref[3 * HID:4 * HID, :]
        w1e = relw1_ref[4 * HID:4 * HID + DE, :]
        ph = (jnp.dot(hi, w1a, preferred_element_type=jnp.float32)
              + jnp.dot(hj, w1b, preferred_element_type=jnp.float32)
              + jnp.dot(hi * hj, w1c, preferred_element_type=jnp.float32)
              + jnp.dot(jnp.abs(hi - hj), w1d, preferred_element_type=jnp.float32)
              + lax.dot_general(fec_ref[:, :], w1e, (((0,), (0,)), ((), ())),
                                preferred_element_type=jnp.float32)
              + relb1_ref[:, :])
        rh = jnp.maximum(ph, 0.0)
        rel_ref[:, :] = jnp.dot(rh, relw2_ref[:, :],
                                preferred_element_type=jnp.float32) + relb2_ref[:, :]

        keys = jnp.dot(h, attnk_ref[:, :], preferred_element_type=jnp.float32)
        al = jnp.dot(keys, attnq_ref[:, :],
                     preferred_element_type=jnp.float32) * scale
        m = jnp.max(al)
        e = jnp.exp(al - m)
        attn = e / jnp.sum(e)
        pooled = jnp.sum(attn * h, axis=0, keepdims=True)
        g = jnp.dot(pooled, gpw_ref[:, :],
                    preferred_element_type=jnp.float32) + gpb_ref[:, :]
        g_ref[:, :] = g
        role_ref[:, :] = jnp.dot(g, rolew_ref[:, :],
                                 preferred_element_type=jnp.float32) + roleb_ref[:, :]
        cnt_ref[:, :] = jnp.dot(g, cntw_ref[:, :],
                                preferred_element_type=jnp.float32) + cntb_ref[:, :]
        noop_ref[:, :] = jnp.dot(g, noopw_ref[:, :],
                                 preferred_element_type=jnp.float32) + noopb_ref[:, :]


def _full(shape):
    zeros = (0,) * len(shape)
    return pl.BlockSpec(shape, lambda b, z=zeros: z)


@jax.jit
def _run(node_embeddings, ef_t, adjacency, wcs, wct, wrs, wrt,
         w1bd, b1big, w2big, b2pair, relw1, relb1, relw2, relb2,
         attnk, attnq, gpw, gpb, rolew, roleb, cntw, cntb, noopw, noopb):
    out_shapes = (
        jax.ShapeDtypeStruct((N, N), jnp.float32),
        jax.ShapeDtypeStruct((N, N), jnp.float32),
        jax.ShapeDtypeStruct((1, N_REL), jnp.float32),
        jax.ShapeDtypeStruct((1, N_ROLE), jnp.float32),
        jax.ShapeDtypeStruct((1, N_CNT), jnp.float32),
        jax.ShapeDtypeStruct((1, 1), jnp.float32),
        jax.ShapeDtypeStruct((1, HID), jnp.float32),
    )
    in_specs = [
        _full((N, HID)),
        pl.BlockSpec((BLK * DE, N), lambda b: (b, 0)),
        pl.BlockSpec((BLK, N), lambda b: (b, 0)),
        _full((HID, HID)), _full((HID, HID)), _full((HID, HID)), _full((HID, HID)),
        _full((2 * CH * HID, CH * DE)), _full((2 * CH * HID, 1)),
        _full((2 * CH * HID, 1)), _full((1, 2)),
        _full((4 * HID + DE, HID)), _full((1, HID)),
        _full((HID, N_REL)), _full((1, N_REL)),
        _full((HID, HID)), _full((HID, 1)),
        _full((HID, HID)), _full((1, HID)),
        _full((HID, N_ROLE)), _full((1, N_ROLE)),
        _full((HID, N_CNT)), _full((1, N_CNT)),
        _full((HID, 1)), _full((1, 1)),
    ]
    out_specs = (
        pl.BlockSpec((BLK, N), lambda b: (b, 0)),
        pl.BlockSpec((BLK, N), lambda b: (b, 0)),
        _full((1, N_REL)), _full((1, N_ROLE)), _full((1, N_CNT)),
        _full((1, 1)), _full((1, HID)),
    )
    return pl.pallas_call(
        _fused_kernel,
        grid=(NB,),
        in_specs=in_specs,
        out_specs=out_specs,
        out_shape=out_shapes,
        scratch_shapes=[
            pltpu.SMEM((1,), jnp.float32),
            pltpu.SMEM((2,), jnp.int32),
            pltpu.VMEM((DE, 1), jnp.float32),
        ],
        compiler_params=pltpu.CompilerParams(
            dimension_semantics=("arbitrary",),
        ),
    )(node_embeddings, ef_t, adjacency, wcs, wct, wrs, wrt,
      w1bd, b1big, w2big, b2pair, relw1, relb1, relw2, relb2,
      attnk, attnq, gpw, gpb, rolew, roleb, cntw, cntb, noopw, noopb)


def kernel(node_embeddings, edge_features, adjacency, params):
    p = params
    ef_t = jnp.transpose(edge_features,
                         (0, 2, 1)).reshape(N * DE, N)
    eye_ch = jnp.eye(CH, dtype=jnp.float32)
    w1bd = jnp.concatenate([
        jnp.kron(eye_ch, p["ceb_w1"].T),
        jnp.kron(eye_ch, p["reb_w1"].T),
    ], axis=0)
    b1big = jnp.concatenate([
        jnp.tile(p["ceb_b1"], CH), jnp.tile(p["reb_b1"], CH),
    ]).reshape(2 * CH * HID, 1)
    w2big = jnp.concatenate([
        jnp.tile(p["ceb_w2"][:, 0], CH), jnp.tile(p["reb_w2"][:, 0], CH),
    ]).reshape(2 * CH * HID, 1)
    b2pair = jnp.stack([p["ceb_b2"][0], p["reb_b2"][0]]).reshape(1, 2)

    outs = _run(node_embeddings, ef_t, adjacency,
                p["W_cs"], p["W_ct"], p["W_rs"], p["W_rt"],
                w1bd, b1big, w2big, b2pair,
                p["rel_w1"], p["rel_b1"].reshape(1, HID),
                p["rel_w2"], p["rel_b2"].reshape(1, N_REL),
                p["attn_k"], p["attn_q"].reshape(HID, 1),
                p["gp_w"], p["gp_b"].reshape(1, HID),
                p["role_w"], p["role_b"].reshape(1, N_ROLE),
                p["cnt_w"], p["cnt_b"].reshape(1, N_CNT),
                p["noop_w"], p["noop_b"].reshape(1, 1))
    create, remove, rel, role, cnt, noop, g = outs
    return (create, remove, rel.reshape(N_REL), role.reshape(N_ROLE),
            cnt.reshape(N_CNT), noop.reshape(1), g.reshape(HID))


# speedup vs baseline: 1.4250x; 1.0090x over previous
import jax
import jax.numpy as jnp
from jax import lax
from jax.experimental import pallas as pl
from jax.experimental.pallas import tpu as pltpu

N = 1024
HID = 128
DE = 8
BLK = 128
NB = N // BLK
CH = 16
NCH = BLK // CH
N_REL = 12
N_ROLE = 16
N_CNT = 9


def _fused_kernel(h_ref, eft_ref, adj_ref,
                  wcs_ref, wct_ref, wrs_ref, wrt_ref,
                  w1bd_ref, b1big_ref, w2big_ref, b2pair_ref,
                  relw1_ref, relb1_ref, relw2_ref, relb2_ref,
                  attnk_ref, attnq_ref,
                  gpw_ref, gpb_ref, rolew_ref, roleb_ref,
                  cntw_ref, cntb_ref, noopw_ref, noopb_ref,
                  create_ref, remove_ref, rel_ref, role_ref,
                  cnt_ref, noop_ref, g_ref,
                  smax_ref, sidx_ref, fec_ref):
    b = pl.program_id(0)
    scale = 1.0 / jnp.sqrt(jnp.float32(HID))
    h = h_ref[:, :]
    hb = h_ref[pl.ds(b * BLK, BLK), :]

    ac = jnp.dot(hb, wcs_ref[:, :], preferred_element_type=jnp.float32)
    bc = jnp.dot(h, wct_ref[:, :], preferred_element_type=jnp.float32)
    sc = lax.dot_general(ac, bc, (((1,), (1,)), ((), ())),
                         preferred_element_type=jnp.float32) * scale
    ar = jnp.dot(hb, wrs_ref[:, :], preferred_element_type=jnp.float32)
    br = jnp.dot(h, wrt_ref[:, :], preferred_element_type=jnp.float32)
    sr = lax.dot_general(ar, br, (((1,), (1,)), ((), ())),
                         preferred_element_type=jnp.float32) * scale
    create_ref[:, :] = sc
    remove_ref[:, :] = sr

    b2c = b2pair_ref[0, 0]
    b2r = b2pair_ref[0, 1]

    def chunk_body(c, carry):
        x = eft_ref[pl.ds(c * CH * DE, CH * DE), :]
        hid = jnp.dot(w1bd_ref[:, :], x,
                      preferred_element_type=jnp.float32) + b1big_ref[:, :]
        hid = jnp.maximum(hid, 0.0) * w2big_ref[:, :]
        srows = hid.reshape(2 * CH, HID, N).sum(axis=1)
        create_ref[pl.ds(c * CH, CH), :] += srows[0:CH] + b2c
        remove_ref[pl.ds(c * CH, CH), :] += srows[CH:2 * CH] + b2r
        return carry

    lax.fori_loop(0, NCH, chunk_body, 0)

    rows_l = lax.broadcasted_iota(jnp.int32, (BLK, N), 0)
    cols = lax.broadcasted_iota(jnp.int32, (BLK, N), 1)
    rows_g = rows_l + b * BLK
    valid = (adj_ref[:, :] <= 0.5) & (rows_g != cols)
    mc = jnp.where(valid, create_ref[:, :], -jnp.inf)
    vm = jnp.max(mc)
    flat = rows_g * N + cols
    cand = jnp.where(mc == vm, flat, jnp.int32(2**31 - 1))
    li = jnp.min(cand)

    @pl.when((vm > smax_ref[0]) | (b == 0))
    def _update():
        smax_ref[0] = vm
        ii = li // N
        jj = li - ii * N
        sidx_ref[0] = ii
        sidx_ref[1] = jj
        i_loc = ii - b * BLK
        slab = eft_ref[pl.ds(i_loc * DE, DE), :]
        colmask = lax.broadcasted_iota(jnp.int32, (DE, N), 1) == jj
        fec_ref[:, :] = jnp.sum(jnp.where(colmask, slab, 0.0), axis=1,
                                keepdims=True)

    @pl.when(b == NB - 1)
    def _finish():
        ii = sidx_ref[0]
        jj = sidx_ref[1]
        hi = h_ref[pl.ds(ii, 1), :]
        hj = h_ref[pl.ds(jj, 1), :]
        w1a = relw1_ref[0:HID, :]
        w1b = relw1_ref[HID:2 * HID, :]
        w1c = relw1_ref[2 * HID:3 * HID, :]
        w1d = relw1_ref[3 * HID:4 * HID, :]
        w1e = relw1_ref[4 * HID:4 * HID + DE, :]
        ph = (jnp.dot(hi, w1a, preferred_element_type=jnp.float32)
              + jnp.dot(hj, w1b, preferred_element_type=jnp.float32)
              + jnp.dot(hi * hj, w1c, preferred_element_type=jnp.float32)
              + jnp.dot(jnp.abs(hi - hj), w1d, preferred_element_type=jnp.float32)
              + lax.dot_general(fec_ref[:, :], w1e, (((0,), (0,)), ((), ())),
                                preferred_element_type=jnp.float32)
              + relb1_ref[:, :])
        rh = jnp.maximum(ph, 0.0)
        rel_ref[:, :] = jnp.dot(rh, relw2_ref[:, :],
                                preferred_element_type=jnp.float32) + relb2_ref[:, :]

        keys = jnp.dot(h, attnk_ref[:, :], preferred_element_type=jnp.float32)
        al = jnp.dot(keys, attnq_ref[:, :],
                     preferred_element_type=jnp.float32) * scale
        m = jnp.max(al)
        e = jnp.exp(al - m)
        attn = e / jnp.sum(e)
        pooled = jnp.sum(attn * h, axis=0, keepdims=True)
        g = jnp.dot(pooled, gpw_ref[:, :],
                    preferred_element_type=jnp.float32) + gpb_ref[:, :]
        g_ref[:, :] = g
        role_ref[:, :] = jnp.dot(g, rolew_ref[:, :],
                                 preferred_element_type=jnp.float32) + roleb_ref[:, :]
        cnt_ref[:, :] = jnp.dot(g, cntw_ref[:, :],
                                preferred_element_type=jnp.float32) + cntb_ref[:, :]
        noop_ref[:, :] = jnp.sum(g * noopw_ref[:, :], axis=1,
                                 keepdims=True) + noopb_ref[:, :]


def _full(shape):
    zeros = (0,) * len(shape)
    return pl.BlockSpec(shape, lambda b, z=zeros: z)


@jax.jit
def _run(node_embeddings, ef_t, adjacency, wcs, wct, wrs, wrt,
         w1bd, b1big, w2big, b2pair, relw1, relb1, relw2, relb2,
         attnk, attnq, gpw, gpb, rolew, roleb, cntw, cntb, noopw, noopb):
    out_shapes = (
        jax.ShapeDtypeStruct((N, N), jnp.float32),
        jax.ShapeDtypeStruct((N, N), jnp.float32),
        jax.ShapeDtypeStruct((1, N_REL), jnp.float32),
        jax.ShapeDtypeStruct((1, N_ROLE), jnp.float32),
        jax.ShapeDtypeStruct((1, N_CNT), jnp.float32),
        jax.ShapeDtypeStruct((1, 1), jnp.float32),
        jax.ShapeDtypeStruct((1, HID), jnp.float32),
    )
    in_specs = [
        _full((N, HID)),
        pl.BlockSpec((BLK * DE, N), lambda b: (b, 0)),
        pl.BlockSpec((BLK, N), lambda b: (b, 0)),
        _full((HID, HID)), _full((HID, HID)), _full((HID, HID)), _full((HID, HID)),
        _full((2 * CH * HID, CH * DE)), _full((2 * CH * HID, 1)),
        _full((2 * CH * HID, 1)), _full((1, 2)),
        _full((4 * HID + DE, HID)), _full((1, HID)),
        _full((HID, N_REL)), _full((1, N_REL)),
        _full((HID, HID)), _full((HID, 1)),
        _full((HID, HID)), _full((1, HID)),
        _full((HID, N_ROLE)), _full((1, N_ROLE)),
        _full((HID, N_CNT)), _full((1, N_CNT)),
        _full((1, HID)), _full((1, 1)),
    ]
    out_specs = (
        pl.BlockSpec((BLK, N), lambda b: (b, 0)),
        pl.BlockSpec((BLK, N), lambda b: (b, 0)),
        _full((1, N_REL)), _full((1, N_ROLE)), _full((1, N_CNT)),
        _full((1, 1)), _full((1, HID)),
    )
    return pl.pallas_call(
        _fused_kernel,
        grid=(NB,),
        in_specs=in_specs,
        out_specs=out_specs,
        out_shape=out_shapes,
        scratch_shapes=[
            pltpu.SMEM((1,), jnp.float32),
            pltpu.SMEM((2,), jnp.int32),
            pltpu.VMEM((DE, 1), jnp.float32),
        ],
        compiler_params=pltpu.CompilerParams(
            dimension_semantics=("arbitrary",),
        ),
    )(node_embeddings, ef_t, adjacency, wcs, wct, wrs, wrt,
      w1bd, b1big, w2big, b2pair, relw1, relb1, relw2, relb2,
      attnk, attnq, gpw, gpb, rolew, roleb, cntw, cntb, noopw, noopb)


def kernel(node_embeddings, edge_features, adjacency, params):
    p = params
    ef_t = jnp.transpose(edge_features,
                         (0, 2, 1)).reshape(N * DE, N)
    eye_ch = jnp.eye(CH, dtype=jnp.float32)
    w1bd = jnp.concatenate([
        jnp.kron(eye_ch, p["ceb_w1"].T),
        jnp.kron(eye_ch, p["reb_w1"].T),
    ], axis=0)
    b1big = jnp.concatenate([
        jnp.tile(p["ceb_b1"], CH), jnp.tile(p["reb_b1"], CH),
    ]).reshape(2 * CH * HID, 1)
    w2big = jnp.concatenate([
        jnp.tile(p["ceb_w2"][:, 0], CH), jnp.tile(p["reb_w2"][:, 0], CH),
    ]).reshape(2 * CH * HID, 1)
    b2pair = jnp.stack([p["ceb_b2"][0], p["reb_b2"][0]]).reshape(1, 2)

    outs = _run(node_embeddings, ef_t, adjacency,
                p["W_cs"], p["W_ct"], p["W_rs"], p["W_rt"],
                w1bd, b1big, w2big, b2pair,
                p["rel_w1"], p["rel_b1"].reshape(1, HID),
                p["rel_w2"], p["rel_b2"].reshape(1, N_REL),
                p["attn_k"], p["attn_q"].reshape(HID, 1),
                p["gp_w"], p["gp_b"].reshape(1, HID),
                p["role_w"], p["role_b"].reshape(1, N_ROLE),
                p["cnt_w"], p["cnt_b"].reshape(1, N_CNT),
                p["noop_w"].reshape(1, HID), p["noop_b"].reshape(1, 1))
    create, remove, rel, role, cnt, noop, g = outs
    return (create, remove, rel.reshape(N_REL), role.reshape(N_ROLE),
            cnt.reshape(N_CNT), noop.reshape(1), g.reshape(HID))
